# baseline (speedup 1.0000x reference)
import functools

import jax
import jax.numpy as jnp
import numpy as np
from jax import lax
from jax.experimental import pallas as pl
from jax.experimental.pallas import tpu as pltpu
from jax.experimental.pallas import tpu_sc as plsc

_F = 26
_V = 104000
_D = 16
_B = 4096
_NW = 32

_VCH = 128
_VPAD = 104064
_TCK = _VPAD // _VCH
_FH = _F // 2


@functools.cache
def _make_sc_phase1():
    mesh = plsc.VectorSubcoreMesh(core_axis_name="c", subcore_axis_name="s")

    @functools.partial(
        pl.kernel,
        out_type=jax.ShapeDtypeStruct((_VPAD, _D), jnp.float32),
        mesh=mesh,
        scratch_types=[
            pltpu.VMEM((3, _FH, _D, _VCH), jnp.float32),
            pltpu.VMEM((2, _VCH, _D), jnp.float32),
            pltpu.VMEM((3, _VCH), jnp.float32),
            pltpu.VMEM((_D * _VCH,), jnp.float32),
            pltpu.VMEM((_D * _VCH,), jnp.float32),
            [pltpu.SemaphoreType.DMA] * 3,
            [pltpu.SemaphoreType.DMA] * 2,
        ],
        compiler_params=pltpu.CompilerParams(needs_layout_passes=False),
    )
    def _sc_phase1(tt_hbm, lin_hbm, out_hbm, buf, hv, linv, sts, stq,
                   ssems, hsems):
        c = lax.axis_index("c")
        s = lax.axis_index("s")
        wid = c * 16 + s
        nck = 25 + jnp.where(wid < _TCK % _NW, 1, 0)
        nwv = 2 * nck

        def _tstart(w, sl):
            ch = w // 2
            half = lax.rem(w, 2)
            v0 = (wid + ch * _NW) * _VCH
            f0 = half * _FH
            pltpu.async_copy(
                tt_hbm.at[pl.ds(f0, _FH), :, pl.ds(v0, _VCH)],
                buf.at[sl], ssems[sl])

            @pl.when(half == 0)
            def _():
                pltpu.async_copy(
                    lin_hbm.at[pl.ds(v0, _VCH)], linv.at[sl], ssems[sl])

        def _acc(w, sl):
            half = lax.rem(w, 2)

            @pl.when(half == 0)
            def _():
                def _d0(d, carry):
                    for g in range(_VCH // _D):
                        acc = jnp.zeros((_D,), jnp.float32)
                        sq = jnp.zeros((_D,), jnp.float32)
                        for f in range(_FH):
                            v = buf[sl, f, d, pl.ds(g * _D, _D)]
                            acc = acc + v
                            sq = sq + v * v
                        sts[pl.ds(d * _VCH + g * _D, _D)] = acc
                        stq[pl.ds(d * _VCH + g * _D, _D)] = sq
                    return carry
                lax.fori_loop(0, _D, _d0, 0)

            @pl.when(half == 1)
            def _():
                ch = w // 2
                v0 = (wid + ch * _NW) * _VCH
                lsl = (sl + 2) % 3

                def _d1(d, carry):
                    for g in range(_VCH // _D):
                        p = pl.ds(d * _VCH + g * _D, _D)
                        acc = sts[p]
                        sq = stq[p]
                        for f in range(_FH):
                            v = buf[sl, f, d, pl.ds(g * _D, _D)]
                            acc = acc + v
                            sq = sq + v * v
                        lvec = linv[lsl, pl.ds(g * _D, _D)]
                        sts[p] = 0.5 * (acc * acc - sq) + lvec
                    return carry
                lax.fori_loop(0, _D, _d1, 0)

                hb = lax.rem(ch, 2)
                for hh in range(2):
                    @pl.when((ch >= 2) & (hb == hh))
                    def _(hh=hh):
                        pltpu.make_async_copy(
                            hv.at[hh], out_hbm.at[pl.ds(0, _VCH)], hsems[hh]
                        ).wait()

                def _tp(vv, carry2):
                    idxv = lax.iota(jnp.int32, 16) * _VCH + vv
                    hv[hb, vv, :] = plsc.load_gather(sts, [idxv])
                    return carry2
                lax.fori_loop(0, _VCH, _tp, 0)

                for hh in range(2):
                    @pl.when(hb == hh)
                    def _(hh=hh):
                        pltpu.async_copy(
                            hv.at[hh], out_hbm.at[pl.ds(v0, _VCH)],
                            hsems[hh])

        for sl in range(3):
            _tstart(sl, sl)

        def _stepw(w, carry):
            wsel = lax.rem(w, 3)

            def _one(sl):
                pltpu.make_async_copy(
                    tt_hbm.at[pl.ds(0, _FH), :, pl.ds(0, _VCH)],
                    buf.at[sl], ssems[sl]).wait()

                @pl.when(lax.rem(w, 2) == 0)
                def _():
                    pltpu.make_async_copy(
                        lin_hbm.at[pl.ds(0, _VCH)], linv.at[sl],
                        ssems[sl]).wait()

                _acc(w, sl)

                @pl.when(w + 3 < nwv)
                def _():
                    _tstart(w + 3, sl)

            for ss in range(3):
                @pl.when(wsel == ss)
                def _(ss=ss):
                    _one(ss)
            return carry
        lax.fori_loop(0, nwv, _stepw, 0)

        for hh in range(2):
            pltpu.make_async_copy(
                hv.at[hh], out_hbm.at[pl.ds(0, _VCH)], hsems[hh]).wait()

    return _sc_phase1


_ROWS = _B * _F
_BPW = _ROWS // _NW
_CH = 128
_NCH = _BPW // _CH
_HALF = _NCH // 2


@functools.cache
def _make_sc_gather():
    mesh = plsc.VectorSubcoreMesh(core_axis_name="c", subcore_axis_name="s")

    @functools.partial(
        pl.kernel,
        out_type=jax.ShapeDtypeStruct((_F, _D, _B), jnp.float32),
        mesh=mesh,
        scratch_types=[
            pltpu.VMEM((_NCH, _CH), jnp.int32),
            pltpu.VMEM((_HALF * _CH, _D), jnp.float32),
            pltpu.VMEM((_HALF, _D, _CH), jnp.float32),
            pltpu.SemaphoreType.DMA,
            pltpu.SemaphoreType.DMA,
        ],
        compiler_params=pltpu.CompilerParams(
            use_tc_tiling_on_sc=False, needs_layout_passes=False),
    )
    def _sc_gather(h_hbm, idx_hbm, out_hbm, idx_v, rows_v, outt, sem, wsem):
        wid = lax.axis_index("s") * 2 + lax.axis_index("c")
        b0 = wid * _CH
        pltpu.sync_copy(idx_hbm.at[:, wid, :], idx_v)

        iot = lax.iota(jnp.int32, 16)
        for half in range(2):
            f0 = half * _HALF
            cps = []
            for jj in range(_HALF):
                cps.append(
                    pltpu.async_copy(
                        h_hbm.at[idx_v.at[f0 + jj]],
                        rows_v.at[pl.ds(jj * _CH, _CH)],
                        sem,
                    )
                )
            if half == 1:
                pltpu.make_async_copy(
                    outt, out_hbm.at[pl.ds(0, _HALF), :, pl.ds(0, _CH)],
                    wsem).wait()

            for jj in range(_HALF):
                cps[jj].wait()

                def _tp(d, carry, jj=jj):
                    for g in range(_CH // _D):
                        ivec = jj * _CH + g * _D + iot
                        dvec = jnp.full((_D,), d, jnp.int32)
                        outt[jj, d, pl.ds(g * _D, _D)] = plsc.load_gather(
                            rows_v, [ivec, dvec])
                    return carry
                lax.fori_loop(0, _D, _tp, 0)

            pltpu.async_copy(
                outt, out_hbm.at[pl.ds(f0, _HALF), :, pl.ds(b0, _CH)], wsem)
        pltpu.make_async_copy(
            outt, out_hbm.at[pl.ds(0, _HALF), :, pl.ds(0, _CH)], wsem).wait()

    return _sc_gather


_OFFSETS = np.arange(_F, dtype=np.int32) * 4000


def kernel(x, tables, linear_w, bias):
    tt = jnp.transpose(tables, (0, 2, 1))
    linp = jnp.pad(linear_w.reshape(_V) + bias[0], (0, _VPAD - _V))
    h = _make_sc_phase1()(tt, linp)
    idxT = (x + jnp.asarray(_OFFSETS)[None, :]).T.reshape(_F, _NW, _CH)
    out3 = _make_sc_gather()(h, idxT)
    return jnp.transpose(out3, (2, 0, 1))


# speedup vs baseline: 1.0067x; 1.0067x over previous
import functools

import jax
import jax.numpy as jnp
import numpy as np
from jax import lax
from jax.experimental import pallas as pl
from jax.experimental.pallas import tpu as pltpu
from jax.experimental.pallas import tpu_sc as plsc

_F = 26
_V = 104000
_D = 16
_B = 4096
_NW = 32

_VCH = 128
_VPAD = 104064
_TCK = _VPAD // _VCH
_FH = _F // 2


@functools.cache
def _make_sc_phase1():
    mesh = plsc.VectorSubcoreMesh(core_axis_name="c", subcore_axis_name="s")

    @functools.partial(
        pl.kernel,
        out_type=jax.ShapeDtypeStruct((_VPAD, _D), jnp.float32),
        mesh=mesh,
        scratch_types=[
            pltpu.VMEM((3, _FH, _D, _VCH), jnp.float32),
            pltpu.VMEM((2, _VCH, _D), jnp.float32),
            pltpu.VMEM((3, _VCH), jnp.float32),
            pltpu.VMEM((_D * _VCH,), jnp.float32),
            pltpu.VMEM((_D * _VCH,), jnp.float32),
            [pltpu.SemaphoreType.DMA] * 3,
            [pltpu.SemaphoreType.DMA] * 2,
        ],
        compiler_params=pltpu.CompilerParams(needs_layout_passes=False),
    )
    def _sc_phase1(tt_hbm, lin_hbm, out_hbm, buf, hv, linv, sts, stq,
                   ssems, hsems):
        c = lax.axis_index("c")
        s = lax.axis_index("s")
        wid = c * 16 + s
        nck = 25 + jnp.where(wid < _TCK % _NW, 1, 0)
        nwv = 2 * nck

        def _tstart(w, sl):
            ch = w // 2
            half = lax.rem(w, 2)
            v0 = (wid + ch * _NW) * _VCH
            f0 = half * _FH
            pltpu.async_copy(
                tt_hbm.at[pl.ds(f0, _FH), :, pl.ds(v0, _VCH)],
                buf.at[sl], ssems[sl])

            @pl.when(half == 0)
            def _():
                pltpu.async_copy(
                    lin_hbm.at[pl.ds(v0, _VCH)], linv.at[sl], ssems[sl])

        def _acc(w, sl):
            half = lax.rem(w, 2)

            @pl.when(half == 0)
            def _():
                def _d0(d, carry):
                    for g in range(_VCH // _D):
                        acc = jnp.zeros((_D,), jnp.float32)
                        sq = jnp.zeros((_D,), jnp.float32)
                        for f in range(_FH):
                            v = buf[sl, f, d, pl.ds(g * _D, _D)]
                            acc = acc + v
                            sq = sq + v * v
                        sts[pl.ds(d * _VCH + g * _D, _D)] = acc
                        stq[pl.ds(d * _VCH + g * _D, _D)] = sq
                    return carry
                lax.fori_loop(0, _D, _d0, 0)

            @pl.when(half == 1)
            def _():
                ch = w // 2
                v0 = (wid + ch * _NW) * _VCH
                lsl = (sl + 2) % 3

                def _d1(d, carry):
                    for g in range(_VCH // _D):
                        p = pl.ds(d * _VCH + g * _D, _D)
                        acc = sts[p]
                        sq = stq[p]
                        for f in range(_FH):
                            v = buf[sl, f, d, pl.ds(g * _D, _D)]
                            acc = acc + v
                            sq = sq + v * v
                        lvec = linv[lsl, pl.ds(g * _D, _D)]
                        sts[p] = 0.5 * (acc * acc - sq) + lvec
                    return carry
                lax.fori_loop(0, _D, _d1, 0)

                hb = lax.rem(ch, 2)
                for hh in range(2):
                    @pl.when((ch >= 2) & (hb == hh))
                    def _(hh=hh):
                        pltpu.make_async_copy(
                            hv.at[hh], out_hbm.at[pl.ds(0, _VCH)], hsems[hh]
                        ).wait()

                def _tp(vv, carry2):
                    idxv = lax.iota(jnp.int32, 16) * _VCH + vv
                    hv[hb, vv, :] = plsc.load_gather(sts, [idxv])
                    return carry2
                lax.fori_loop(0, _VCH, _tp, 0)

                for hh in range(2):
                    @pl.when(hb == hh)
                    def _(hh=hh):
                        pltpu.async_copy(
                            hv.at[hh], out_hbm.at[pl.ds(v0, _VCH)],
                            hsems[hh])

        for sl in range(3):
            _tstart(sl, sl)

        def _stepw(w, carry):
            wsel = lax.rem(w, 3)

            def _one(sl):
                pltpu.make_async_copy(
                    tt_hbm.at[pl.ds(0, _FH), :, pl.ds(0, _VCH)],
                    buf.at[sl], ssems[sl]).wait()

                @pl.when(lax.rem(w, 2) == 0)
                def _():
                    pltpu.make_async_copy(
                        lin_hbm.at[pl.ds(0, _VCH)], linv.at[sl],
                        ssems[sl]).wait()

                _acc(w, sl)

                @pl.when(w + 3 < nwv)
                def _():
                    _tstart(w + 3, sl)

            for ss in range(3):
                @pl.when(wsel == ss)
                def _(ss=ss):
                    _one(ss)
            return carry
        lax.fori_loop(0, nwv, _stepw, 0)

        for hh in range(2):
            pltpu.make_async_copy(
                hv.at[hh], out_hbm.at[pl.ds(0, _VCH)], hsems[hh]).wait()

    return _sc_phase1


_ROWS = _B * _F
_BPW = _ROWS // _NW
_CH = 128
_NCH = _BPW // _CH
_HALF = _NCH // 2


@functools.cache
def _make_sc_gather():
    mesh = plsc.VectorSubcoreMesh(core_axis_name="c", subcore_axis_name="s")

    @functools.partial(
        pl.kernel,
        out_type=jax.ShapeDtypeStruct((_F, _D, _B), jnp.float32),
        mesh=mesh,
        scratch_types=[
            pltpu.VMEM((_NCH, _CH), jnp.int32),
            pltpu.VMEM((_HALF * _CH, _D), jnp.float32),
            pltpu.VMEM((_HALF, _D, _CH), jnp.float32),
            pltpu.SemaphoreType.DMA,
            pltpu.SemaphoreType.DMA,
        ],
        compiler_params=pltpu.CompilerParams(
            use_tc_tiling_on_sc=False, needs_layout_passes=False),
    )
    def _sc_gather(h_hbm, idx_hbm, out_hbm, idx_v, rows_v, outt, sem, wsem):
        wid = lax.axis_index("s") * 2 + lax.axis_index("c")
        b0 = wid * _CH
        pltpu.sync_copy(idx_hbm.at[:, wid, :], idx_v)

        for half in range(2):
            f0 = half * _HALF
            cps = []
            for jj in range(_HALF):
                cps.append(
                    pltpu.async_copy(
                        h_hbm.at[idx_v.at[f0 + jj]],
                        rows_v.at[pl.ds(jj * _CH, _CH)],
                        sem,
                    )
                )
            for cp in cps:
                cp.wait()

            if half == 1:
                pltpu.make_async_copy(
                    outt, out_hbm.at[pl.ds(0, _HALF), :, pl.ds(0, _CH)],
                    wsem).wait()

            def _tp(jd, carry):
                j = jd // _D
                d = lax.rem(jd, _D)
                for g in range(_CH // _D):
                    ivec = j * _CH + g * _D + lax.iota(jnp.int32, 16)
                    dvec = jnp.full((_D,), d, jnp.int32)
                    outt[j, d, pl.ds(g * _D, _D)] = plsc.load_gather(
                        rows_v, [ivec, dvec])
                return carry
            lax.fori_loop(0, _HALF * _D, _tp, 0)

            pltpu.async_copy(
                outt, out_hbm.at[pl.ds(f0, _HALF), :, pl.ds(b0, _CH)], wsem)
        pltpu.make_async_copy(
            outt, out_hbm.at[pl.ds(0, _HALF), :, pl.ds(0, _CH)], wsem).wait()

    return _sc_gather


_OFFSETS = np.arange(_F, dtype=np.int32) * 4000


def kernel(x, tables, linear_w, bias):
    tt = jnp.transpose(tables, (0, 2, 1))
    linp = jnp.pad(linear_w.reshape(_V) + bias[0], (0, _VPAD - _V))
    h = _make_sc_phase1()(tt, linp)
    idxT = (x + jnp.asarray(_OFFSETS)[None, :]).T.reshape(_F, _NW, _CH)
    out3 = _make_sc_gather()(h, idxT)
    return jnp.transpose(out3, (2, 0, 1))


# speedup vs baseline: 1.0165x; 1.0097x over previous
import functools

import jax
import jax.numpy as jnp
import numpy as np
from jax import lax
from jax.experimental import pallas as pl
from jax.experimental.pallas import tpu as pltpu
from jax.experimental.pallas import tpu_sc as plsc

_F = 26
_V = 104000
_D = 16
_B = 4096
_NW = 32

_VCH = 128
_VPAD = 104064
_TCK = _VPAD // _VCH
_FH = _F // 2


@functools.cache
def _make_sc_phase1():
    mesh = plsc.VectorSubcoreMesh(core_axis_name="c", subcore_axis_name="s")

    @functools.partial(
        pl.kernel,
        out_type=jax.ShapeDtypeStruct((_VPAD, _D), jnp.float32),
        mesh=mesh,
        scratch_types=[
            pltpu.VMEM((3, _FH, _D, _VCH), jnp.float32),
            pltpu.VMEM((2, _VCH, _D), jnp.float32),
            pltpu.VMEM((3, _VCH), jnp.float32),
            pltpu.VMEM((_D * _VCH,), jnp.float32),
            pltpu.VMEM((_D * _VCH,), jnp.float32),
            [pltpu.SemaphoreType.DMA] * 3,
            [pltpu.SemaphoreType.DMA] * 2,
        ],
        compiler_params=pltpu.CompilerParams(needs_layout_passes=False),
    )
    def _sc_phase1(tt_hbm, lin_hbm, out_hbm, buf, hv, linv, sts, stq,
                   ssems, hsems):
        c = lax.axis_index("c")
        s = lax.axis_index("s")
        wid = c * 16 + s
        nck = 25 + jnp.where(wid < _TCK % _NW, 1, 0)
        nwv = 2 * nck

        def _tstart(w, sl):
            ch = w // 2
            half = lax.rem(w, 2)
            v0 = (wid + ch * _NW) * _VCH
            f0 = half * _FH
            pltpu.async_copy(
                tt_hbm.at[pl.ds(f0, _FH), :, pl.ds(v0, _VCH)],
                buf.at[sl], ssems[sl])

            @pl.when(half == 0)
            def _():
                pltpu.async_copy(
                    lin_hbm.at[pl.ds(v0, _VCH)], linv.at[sl], ssems[sl])

        def _acc(w, sl):
            half = lax.rem(w, 2)

            @pl.when(half == 0)
            def _():
                def _d0(d, carry):
                    for g in range(_VCH // _D):
                        acc = jnp.zeros((_D,), jnp.float32)
                        sq = jnp.zeros((_D,), jnp.float32)
                        for f in range(_FH):
                            v = buf[sl, f, d, pl.ds(g * _D, _D)]
                            acc = acc + v
                            sq = sq + v * v
                        sts[pl.ds(d * _VCH + g * _D, _D)] = acc
                        stq[pl.ds(d * _VCH + g * _D, _D)] = sq
                    return carry
                lax.fori_loop(0, _D, _d0, 0)

            @pl.when(half == 1)
            def _():
                ch = w // 2
                v0 = (wid + ch * _NW) * _VCH
                lsl = (sl + 2) % 3

                def _d1(d, carry):
                    for g in range(_VCH // _D):
                        p = pl.ds(d * _VCH + g * _D, _D)
                        acc = sts[p]
                        sq = stq[p]
                        for f in range(_FH):
                            v = buf[sl, f, d, pl.ds(g * _D, _D)]
                            acc = acc + v
                            sq = sq + v * v
                        lvec = linv[lsl, pl.ds(g * _D, _D)]
                        sts[p] = 0.5 * (acc * acc - sq) + lvec
                    return carry
                lax.fori_loop(0, _D, _d1, 0)

                hb = lax.rem(ch, 2)
                for hh in range(2):
                    @pl.when((ch >= 2) & (hb == hh))
                    def _(hh=hh):
                        pltpu.make_async_copy(
                            hv.at[hh], out_hbm.at[pl.ds(0, _VCH)], hsems[hh]
                        ).wait()

                def _tp(vv, carry2):
                    idxv = lax.iota(jnp.int32, 16) * _VCH + vv
                    hv[hb, vv, :] = plsc.load_gather(sts, [idxv])
                    return carry2
                lax.fori_loop(0, _VCH, _tp, 0)

                for hh in range(2):
                    @pl.when(hb == hh)
                    def _(hh=hh):
                        pltpu.async_copy(
                            hv.at[hh], out_hbm.at[pl.ds(v0, _VCH)],
                            hsems[hh])

        for sl in range(3):
            _tstart(sl, sl)

        def _stepw(w, carry):
            wsel = lax.rem(w, 3)

            def _one(sl):
                pltpu.make_async_copy(
                    tt_hbm.at[pl.ds(0, _FH), :, pl.ds(0, _VCH)],
                    buf.at[sl], ssems[sl]).wait()

                @pl.when(lax.rem(w, 2) == 0)
                def _():
                    pltpu.make_async_copy(
                        lin_hbm.at[pl.ds(0, _VCH)], linv.at[sl],
                        ssems[sl]).wait()

                _acc(w, sl)

                @pl.when(w + 3 < nwv)
                def _():
                    _tstart(w + 3, sl)

            for ss in range(3):
                @pl.when(wsel == ss)
                def _(ss=ss):
                    _one(ss)
            return carry
        lax.fori_loop(0, nwv, _stepw, 0)

        for hh in range(2):
            pltpu.make_async_copy(
                hv.at[hh], out_hbm.at[pl.ds(0, _VCH)], hsems[hh]).wait()

    return _sc_phase1


_ROWS = _B * _F
_BPW = _ROWS // _NW
_CH = 128
_NCH = _BPW // _CH
_HALF = _NCH // 2


@functools.cache
def _make_sc_gather():
    mesh = plsc.VectorSubcoreMesh(core_axis_name="c", subcore_axis_name="s")

    @functools.partial(
        pl.kernel,
        out_type=jax.ShapeDtypeStruct((_F, _D, _B), jnp.float32),
        mesh=mesh,
        scratch_types=[
            pltpu.VMEM((_NCH, _CH), jnp.int32),
            pltpu.VMEM((2, _HALF * _CH, _D), jnp.float32),
            pltpu.VMEM((_HALF, _D, _CH), jnp.float32),
            [pltpu.SemaphoreType.DMA] * 2,
            pltpu.SemaphoreType.DMA,
        ],
        compiler_params=pltpu.CompilerParams(
            use_tc_tiling_on_sc=False, needs_layout_passes=False),
    )
    def _sc_gather(h_hbm, idx_hbm, out_hbm, idx_v, rows_v, outt, sems, wsem):
        wid = lax.axis_index("s") * 2 + lax.axis_index("c")
        b0 = wid * _CH
        pltpu.sync_copy(idx_hbm.at[:, wid, :], idx_v)

        def _fire(half):
            f0 = half * _HALF
            return [
                pltpu.async_copy(
                    h_hbm.at[idx_v.at[f0 + jj]],
                    rows_v.at[half, pl.ds(jj * _CH, _CH)],
                    sems[half],
                )
                for jj in range(_HALF)
            ]

        def _transpose_write(half):
            def _tp(jd, carry):
                j = jd // _D
                d = lax.rem(jd, _D)
                for g in range(_CH // _D):
                    hvec = jnp.full((_D,), half, jnp.int32)
                    ivec = j * _CH + g * _D + lax.iota(jnp.int32, 16)
                    dvec = jnp.full((_D,), d, jnp.int32)
                    outt[j, d, pl.ds(g * _D, _D)] = plsc.load_gather(
                        rows_v, [hvec, ivec, dvec])
                return carry
            lax.fori_loop(0, _HALF * _D, _tp, 0)
            pltpu.async_copy(
                outt,
                out_hbm.at[pl.ds(half * _HALF, _HALF), :, pl.ds(b0, _CH)],
                wsem)

        cps_a = _fire(0)
        for cp in cps_a:
            cp.wait()
        cps_b = _fire(1)
        _transpose_write(0)
        for cp in cps_b:
            cp.wait()
        pltpu.make_async_copy(
            outt, out_hbm.at[pl.ds(0, _HALF), :, pl.ds(0, _CH)], wsem).wait()
        _transpose_write(1)
        pltpu.make_async_copy(
            outt, out_hbm.at[pl.ds(0, _HALF), :, pl.ds(0, _CH)], wsem).wait()

    return _sc_gather


_OFFSETS = np.arange(_F, dtype=np.int32) * 4000


def kernel(x, tables, linear_w, bias):
    tt = jnp.transpose(tables, (0, 2, 1))
    linp = jnp.pad(linear_w.reshape(_V) + bias[0], (0, _VPAD - _V))
    h = _make_sc_phase1()(tt, linp)
    idxT = (x + jnp.asarray(_OFFSETS)[None, :]).T.reshape(_F, _NW, _CH)
    out3 = _make_sc_gather()(h, idxT)
    return jnp.transpose(out3, (2, 0, 1))


# speedup vs baseline: 1.0175x; 1.0010x over previous
import functools

import jax
import jax.numpy as jnp
import numpy as np
from jax import lax
from jax.experimental import pallas as pl
from jax.experimental.pallas import tpu as pltpu
from jax.experimental.pallas import tpu_sc as plsc

_F = 26
_V = 104000
_D = 16
_B = 4096
_NW = 32

_VCH = 128
_VPAD = 104064
_TCK = _VPAD // _VCH
_FH = _F // 2


@functools.cache
def _make_sc_phase1():
    mesh = plsc.VectorSubcoreMesh(core_axis_name="c", subcore_axis_name="s")

    @functools.partial(
        pl.kernel,
        out_type=jax.ShapeDtypeStruct((_VPAD, _D), jnp.float32),
        mesh=mesh,
        scratch_types=[
            pltpu.VMEM((3, _FH, _D, _VCH), jnp.float32),
            pltpu.VMEM((2, _VCH, _D), jnp.float32),
            pltpu.VMEM((3, _VCH), jnp.float32),
            pltpu.VMEM((_D * _VCH,), jnp.float32),
            pltpu.VMEM((_D * _VCH,), jnp.float32),
            [pltpu.SemaphoreType.DMA] * 3,
            [pltpu.SemaphoreType.DMA] * 2,
        ],
        compiler_params=pltpu.CompilerParams(needs_layout_passes=False),
    )
    def _sc_phase1(tt_hbm, lin_hbm, out_hbm, buf, hv, linv, sts, stq,
                   ssems, hsems):
        c = lax.axis_index("c")
        s = lax.axis_index("s")
        wid = c * 16 + s
        nck = 25 + jnp.where(wid < _TCK % _NW, 1, 0)
        ck0 = 25 * wid + jnp.minimum(wid, _TCK % _NW)
        nwv = 2 * nck

        def _tstart(w, sl):
            ch = w // 2
            half = lax.rem(w, 2)
            v0 = (ck0 + ch) * _VCH
            f0 = half * _FH
            pltpu.async_copy(
                tt_hbm.at[pl.ds(f0, _FH), :, pl.ds(v0, _VCH)],
                buf.at[sl], ssems[sl])

            @pl.when(half == 0)
            def _():
                pltpu.async_copy(
                    lin_hbm.at[pl.ds(v0, _VCH)], linv.at[sl], ssems[sl])

        def _acc(w, sl):
            half = lax.rem(w, 2)

            @pl.when(half == 0)
            def _():
                def _d0(d, carry):
                    for g in range(_VCH // _D):
                        acc = jnp.zeros((_D,), jnp.float32)
                        sq = jnp.zeros((_D,), jnp.float32)
                        for f in range(_FH):
                            v = buf[sl, f, d, pl.ds(g * _D, _D)]
                            acc = acc + v
                            sq = sq + v * v
                        sts[pl.ds(d * _VCH + g * _D, _D)] = acc
                        stq[pl.ds(d * _VCH + g * _D, _D)] = sq
                    return carry
                lax.fori_loop(0, _D, _d0, 0)

            @pl.when(half == 1)
            def _():
                ch = w // 2
                v0 = (ck0 + ch) * _VCH
                lsl = (sl + 2) % 3

                def _d1(d, carry):
                    for g in range(_VCH // _D):
                        p = pl.ds(d * _VCH + g * _D, _D)
                        acc = sts[p]
                        sq = stq[p]
                        for f in range(_FH):
                            v = buf[sl, f, d, pl.ds(g * _D, _D)]
                            acc = acc + v
                            sq = sq + v * v
                        lvec = linv[lsl, pl.ds(g * _D, _D)]
                        sts[p] = 0.5 * (acc * acc - sq) + lvec
                    return carry
                lax.fori_loop(0, _D, _d1, 0)

                hb = lax.rem(ch, 2)
                for hh in range(2):
                    @pl.when((ch >= 2) & (hb == hh))
                    def _(hh=hh):
                        pltpu.make_async_copy(
                            hv.at[hh], out_hbm.at[pl.ds(0, _VCH)], hsems[hh]
                        ).wait()

                def _tp(vv, carry2):
                    idxv = lax.iota(jnp.int32, 16) * _VCH + vv
                    hv[hb, vv, :] = plsc.load_gather(sts, [idxv])
                    return carry2
                lax.fori_loop(0, _VCH, _tp, 0)

                for hh in range(2):
                    @pl.when(hb == hh)
                    def _(hh=hh):
                        pltpu.async_copy(
                            hv.at[hh], out_hbm.at[pl.ds(v0, _VCH)],
                            hsems[hh])

        for sl in range(3):
            _tstart(sl, sl)

        def _stepw(w, carry):
            wsel = lax.rem(w, 3)

            def _one(sl):
                pltpu.make_async_copy(
                    tt_hbm.at[pl.ds(0, _FH), :, pl.ds(0, _VCH)],
                    buf.at[sl], ssems[sl]).wait()

                @pl.when(lax.rem(w, 2) == 0)
                def _():
                    pltpu.make_async_copy(
                        lin_hbm.at[pl.ds(0, _VCH)], linv.at[sl],
                        ssems[sl]).wait()

                _acc(w, sl)

                @pl.when(w + 3 < nwv)
                def _():
                    _tstart(w + 3, sl)

            for ss in range(3):
                @pl.when(wsel == ss)
                def _(ss=ss):
                    _one(ss)
            return carry
        lax.fori_loop(0, nwv, _stepw, 0)

        for hh in range(2):
            pltpu.make_async_copy(
                hv.at[hh], out_hbm.at[pl.ds(0, _VCH)], hsems[hh]).wait()

    return _sc_phase1


_ROWS = _B * _F
_BPW = _ROWS // _NW
_CH = 128
_NCH = _BPW // _CH
_HALF = _NCH // 2


@functools.cache
def _make_sc_gather():
    mesh = plsc.VectorSubcoreMesh(core_axis_name="c", subcore_axis_name="s")

    @functools.partial(
        pl.kernel,
        out_type=jax.ShapeDtypeStruct((_F, _D, _B), jnp.float32),
        mesh=mesh,
        scratch_types=[
            pltpu.VMEM((_NCH, _CH), jnp.int32),
            pltpu.VMEM((2, _HALF * _CH, _D), jnp.float32),
            pltpu.VMEM((_HALF, _D, _CH), jnp.float32),
            [pltpu.SemaphoreType.DMA] * 2,
            pltpu.SemaphoreType.DMA,
        ],
        compiler_params=pltpu.CompilerParams(
            use_tc_tiling_on_sc=False, needs_layout_passes=False),
    )
    def _sc_gather(h_hbm, idx_hbm, out_hbm, idx_v, rows_v, outt, sems, wsem):
        wid = lax.axis_index("s") * 2 + lax.axis_index("c")
        b0 = wid * _CH
        pltpu.sync_copy(idx_hbm.at[:, wid, :], idx_v)

        def _fire(half):
            f0 = half * _HALF
            return [
                pltpu.async_copy(
                    h_hbm.at[idx_v.at[f0 + jj]],
                    rows_v.at[half, pl.ds(jj * _CH, _CH)],
                    sems[half],
                )
                for jj in range(_HALF)
            ]

        def _transpose_write(half):
            def _tp(jd, carry):
                j = jd // _D
                d = lax.rem(jd, _D)
                for g in range(_CH // _D):
                    hvec = jnp.full((_D,), half, jnp.int32)
                    ivec = j * _CH + g * _D + lax.iota(jnp.int32, 16)
                    dvec = jnp.full((_D,), d, jnp.int32)
                    outt[j, d, pl.ds(g * _D, _D)] = plsc.load_gather(
                        rows_v, [hvec, ivec, dvec])
                return carry
            lax.fori_loop(0, _HALF * _D, _tp, 0)
            pltpu.async_copy(
                outt,
                out_hbm.at[pl.ds(half * _HALF, _HALF), :, pl.ds(b0, _CH)],
                wsem)

        cps_a = _fire(0)
        for cp in cps_a:
            cp.wait()
        cps_b = _fire(1)
        _transpose_write(0)
        for cp in cps_b:
            cp.wait()
        pltpu.make_async_copy(
            outt, out_hbm.at[pl.ds(0, _HALF), :, pl.ds(0, _CH)], wsem).wait()
        _transpose_write(1)
        pltpu.make_async_copy(
            outt, out_hbm.at[pl.ds(0, _HALF), :, pl.ds(0, _CH)], wsem).wait()

    return _sc_gather


_OFFSETS = np.arange(_F, dtype=np.int32) * 4000


def kernel(x, tables, linear_w, bias):
    tt = jnp.transpose(tables, (0, 2, 1))
    linp = jnp.pad(linear_w.reshape(_V) + bias[0], (0, _VPAD - _V))
    h = _make_sc_phase1()(tt, linp)
    idxT = (x + jnp.asarray(_OFFSETS)[None, :]).T.reshape(_F, _NW, _CH)
    out3 = _make_sc_gather()(h, idxT)
    return jnp.transpose(out3, (2, 0, 1))
